# baseline (speedup 1.0000x reference)
import functools

import jax
import jax.numpy as jnp
from jax import lax
from jax.experimental import pallas as pl
from jax.experimental.pallas import tpu as pltpu
from jax.experimental.pallas import tpu_sc as plsc

N_NODES = 10000
N_EDGES = 320000
D = 128

NC = 2
NS = 16
L = 16
NW = NC * NS

CH = 128
NCHUNK = 2560
E_PAD = NCHUNK * CH
CPT = NCHUNK // NS
CPW = NCHUNK // NW
BLK = 16
MBLK = CPW // BLK
DNBLK = CPT // BLK

ROWB = 80
NROWCH = N_NODES // ROWB

MM_BLK = 400
MM_GRID = N_NODES // MM_BLK

_MESH = plsc.VectorSubcoreMesh(core_axis_name="c", subcore_axis_name="s")
_SC_PARAMS = pltpu.CompilerParams(needs_layout_passes=False)


def _gk_body(x_ref, w_ref, d_ref, g_ref, dis_ref):
    d = d_ref[:, 0]
    dis = jnp.where(d > 0.0, lax.rsqrt(jnp.where(d > 0.0, d, 1.0)), 0.0)
    dis_ref[...] = dis[:, None]
    g_ref[...] = jnp.dot(x_ref[...], w_ref[...],
                         preferred_element_type=jnp.float32) * dis[:, None]


def _g_and_dis(x, W, deg):
    return pl.pallas_call(
        _gk_body,
        grid=(MM_GRID,),
        in_specs=[
            pl.BlockSpec((MM_BLK, D), lambda i: (i, 0)),
            pl.BlockSpec((D, D), lambda i: (0, 0)),
            pl.BlockSpec((MM_BLK, 1), lambda i: (i, 0)),
        ],
        out_specs=[
            pl.BlockSpec((MM_BLK, D), lambda i: (i, 0)),
            pl.BlockSpec((MM_BLK, 1), lambda i: (i, 0)),
        ],
        out_shape=[
            jax.ShapeDtypeStruct((N_NODES, D), jnp.float32),
            jax.ShapeDtypeStruct((N_NODES, 1), jnp.float32),
        ],
    )(x, W, deg)


def _fin_body(p_ref, dis_ref, b_ref, o_ref):
    o_ref[...] = (p_ref[0] + p_ref[1]) * dis_ref[...] + b_ref[...]


def _final_add(parts, dis, b):
    return pl.pallas_call(
        _fin_body,
        grid=(MM_GRID,),
        in_specs=[
            pl.BlockSpec((NC, MM_BLK, D), lambda i: (0, i, 0)),
            pl.BlockSpec((MM_BLK, 1), lambda i: (i, 0)),
            pl.BlockSpec((D,), lambda i: (0,)),
        ],
        out_specs=pl.BlockSpec((MM_BLK, D), lambda i: (i, 0)),
        out_shape=jax.ShapeDtypeStruct((N_NODES, D), jnp.float32),
    )(parts, dis, b)


@functools.partial(
    pl.kernel,
    out_type=jax.ShapeDtypeStruct((N_NODES,), jnp.float32),
    mesh=_MESH,
    compiler_params=_SC_PARAMS,
    scratch_types=[
        pltpu.VMEM((BLK, CH), jnp.int32),
        pltpu.VMEM((BLK, CH), jnp.float32),
        pltpu.VMEM((N_NODES,), jnp.float32),
        pltpu.VMEM_SHARED((N_NODES,), jnp.float32),
    ],
)
def _sc_deg(col_hbm, ew_hbm, deg_hbm, colb, ewb, zbuf, deg):
    cid = lax.axis_index("c")
    sid = lax.axis_index("s")

    zv = jnp.zeros((L,), jnp.float32)

    def _zb(i, _):
        zbuf[pl.ds(i * L, L)] = zv
        return 0
    lax.fori_loop(0, N_NODES // L, _zb, 0)

    @pl.when(sid == 0)
    def _():
        pltpu.sync_copy(zbuf, deg)

    plsc.subcore_barrier()

    def _dblk(t, _):
        dbase = sid * CPT + t * BLK
        pltpu.sync_copy(col_hbm.at[pl.ds(dbase, BLK)], colb)
        pltpu.sync_copy(ew_hbm.at[pl.ds(dbase, BLK)], ewb)

        def _dadd(j, _):
            pltpu.sync_copy(ewb.at[j], deg.at[colb.at[j]], add=True)
            return 0
        lax.fori_loop(0, BLK, _dadd, 0)
        return 0
    lax.fori_loop(0, DNBLK, _dblk, 0)

    plsc.subcore_barrier()

    @pl.when(jnp.logical_and(cid == 0, sid == 0))
    def _():
        pltpu.sync_copy(deg, deg_hbm)


@functools.partial(
    pl.kernel,
    out_type=jax.ShapeDtypeStruct((NC, N_NODES, D), jnp.float32),
    mesh=_MESH,
    compiler_params=_SC_PARAMS,
    scratch_types=[
        pltpu.VMEM((BLK, CH), jnp.int32),
        pltpu.VMEM((BLK, CH), jnp.int32),
        pltpu.VMEM((BLK, CH), jnp.float32),
        pltpu.VMEM((CH, D), jnp.float32),
        pltpu.VMEM((CH, D), jnp.float32),
        pltpu.VMEM_SHARED((N_NODES, D), jnp.float32),
        pltpu.SemaphoreType.DMA,
        pltpu.SemaphoreType.DMA,
        pltpu.SemaphoreType.DMA,
        pltpu.SemaphoreType.DMA,
    ],
)
def _sc_msg(g_hbm, row_hbm, col_hbm, ew_hbm, out_hbm,
            rowb, colb, ewb, msgA, msgB, acc, gsA, gsB, ssA, ssB):
    cid = lax.axis_index("c")
    sid = lax.axis_index("s")
    wid = cid * NS + sid

    zv = jnp.zeros((L,), jnp.float32)

    def _zmsg(i, _):
        for k in range(D // L):
            msgA[i, pl.ds(k * L, L)] = zv
        return 0
    lax.fori_loop(0, CH, _zmsg, 0)

    def _zacc(t, _):
        c = sid + t * NS
        @pl.when(c < NROWCH)
        def _():
            r = c * ROWB
            pltpu.sync_copy(msgA.at[pl.ds(0, ROWB)], acc.at[pl.ds(r, ROWB)])
        return 0
    lax.fori_loop(0, (NROWCH + NS - 1) // NS, _zacc, 0)

    plsc.subcore_barrier()

    def _scale(mref, j):
        jv = jnp.full((L,), j, jnp.int32)

        def _rloop(q, _):
            e0 = q * 4
            ns = []
            for u in range(4):
                ev = jnp.full((L,), e0 + u, jnp.int32)
                ns.append(plsc.load_gather(ewb, [jv, ev]))
            for k in range(D // L):
                sl = pl.ds(k * L, L)
                for u in range(4):
                    mref[e0 + u, sl] = mref[e0 + u, sl] * ns[u]
            return 0
        lax.fori_loop(0, CH // 4, _rloop, 0)

    def _mblk(t, _):
        wbase = wid * CPW + t * BLK
        pltpu.sync_copy(row_hbm.at[pl.ds(wbase, BLK)], rowb)
        pltpu.sync_copy(col_hbm.at[pl.ds(wbase, BLK)], colb)
        pltpu.sync_copy(ew_hbm.at[pl.ds(wbase, BLK)], ewb)

        pltpu.async_copy(g_hbm.at[rowb.at[0]], msgA, gsA)

        def _mpair(p, _):
            jA = 2 * p
            jB = 2 * p + 1

            pltpu.make_async_copy(g_hbm.at[rowb.at[jA]], msgA, gsA).wait()
            @pl.when(p > 0)
            def _():
                pltpu.make_async_copy(msgB, acc.at[colb.at[jB - 2]], ssB).wait()
            pltpu.async_copy(g_hbm.at[rowb.at[jB]], msgB, gsB)

            _scale(msgA, jA)
            pltpu.async_copy(msgA, acc.at[colb.at[jA]], ssA, add=True)

            pltpu.make_async_copy(g_hbm.at[rowb.at[jB]], msgB, gsB).wait()
            _scale(msgB, jB)

            @pl.when(p < BLK // 2 - 1)
            def _():
                pltpu.make_async_copy(msgA, acc.at[colb.at[jA]], ssA).wait()
                pltpu.async_copy(g_hbm.at[rowb.at[jA + 2]], msgA, gsA)

            pltpu.async_copy(msgB, acc.at[colb.at[jB]], ssB, add=True)
            return 0
        lax.fori_loop(0, BLK // 2, _mpair, 0)

        pltpu.make_async_copy(msgA, acc.at[colb.at[BLK - 2]], ssA).wait()
        pltpu.make_async_copy(msgB, acc.at[colb.at[BLK - 1]], ssB).wait()
        return 0
    lax.fori_loop(0, MBLK, _mblk, 0)

    plsc.subcore_barrier()

    def _oloop(t, _):
        c = sid + t * NS
        @pl.when(c < NROWCH)
        def _():
            r = c * ROWB
            pltpu.sync_copy(acc.at[pl.ds(r, ROWB)], out_hbm.at[cid, pl.ds(r, ROWB), :])
        return 0
    lax.fori_loop(0, (NROWCH + NS - 1) // NS, _oloop, 0)


def kernel(x, edge_index, edge_weight, W, b):
    pad = E_PAD - N_EDGES
    zi = jnp.arange(pad, dtype=jnp.int32) % N_NODES
    row = jnp.concatenate([edge_index[0].astype(jnp.int32), zi]).reshape(NCHUNK, CH)
    col = jnp.concatenate([edge_index[1].astype(jnp.int32), zi]).reshape(NCHUNK, CH)
    ew = jnp.concatenate([edge_weight, jnp.zeros((pad,), jnp.float32)]).reshape(NCHUNK, CH)
    deg = _sc_deg(col, ew)
    g, dis = _g_and_dis(x, W, deg.reshape(N_NODES, 1))
    parts = _sc_msg(g, row, col, ew)
    return _final_add(parts, dis, b)


# speedup vs baseline: 1.1059x; 1.1059x over previous
import functools

import jax
import jax.numpy as jnp
from jax import lax
from jax.experimental import pallas as pl
from jax.experimental.pallas import tpu as pltpu
from jax.experimental.pallas import tpu_sc as plsc

N_NODES = 10000
N_EDGES = 320000
D = 128

NC = 2
NS = 16
L = 16
NW = NC * NS

CH = 128
NCHUNK = 2560
E_PAD = NCHUNK * CH
CPW = NCHUNK // NW
BLK = 16
MBLK = CPW // BLK
DNBLK = CPW // BLK

ROWB = 80
NROWCH = N_NODES // ROWB

MM_BLK = 400
MM_GRID = N_NODES // MM_BLK

_MESH = plsc.VectorSubcoreMesh(core_axis_name="c", subcore_axis_name="s")
_SC_PARAMS = pltpu.CompilerParams(needs_layout_passes=False)


def _gk_body(x_ref, w_ref, d0_ref, d1_ref, g_ref, dis_ref):
    d = d0_ref[:, 0] + d1_ref[:, 0]
    dis = jnp.where(d > 0.0, lax.rsqrt(jnp.where(d > 0.0, d, 1.0)), 0.0)
    dis_ref[...] = dis[:, None]
    g_ref[...] = jnp.dot(x_ref[...], w_ref[...],
                         preferred_element_type=jnp.float32) * dis[:, None]


def _g_and_dis(x, W, deg0, deg1):
    return pl.pallas_call(
        _gk_body,
        grid=(MM_GRID,),
        in_specs=[
            pl.BlockSpec((MM_BLK, D), lambda i: (i, 0)),
            pl.BlockSpec((D, D), lambda i: (0, 0)),
            pl.BlockSpec((MM_BLK, 1), lambda i: (i, 0)),
            pl.BlockSpec((MM_BLK, 1), lambda i: (i, 0)),
        ],
        out_specs=[
            pl.BlockSpec((MM_BLK, D), lambda i: (i, 0)),
            pl.BlockSpec((MM_BLK, 1), lambda i: (i, 0)),
        ],
        out_shape=[
            jax.ShapeDtypeStruct((N_NODES, D), jnp.float32),
            jax.ShapeDtypeStruct((N_NODES, 1), jnp.float32),
        ],
    )(x, W, deg0, deg1)


def _fin_body(p_ref, dis_ref, b_ref, o_ref):
    o_ref[...] = (p_ref[0] + p_ref[1]) * dis_ref[...] + b_ref[...]


def _final_add(parts, dis, b):
    return pl.pallas_call(
        _fin_body,
        grid=(MM_GRID,),
        in_specs=[
            pl.BlockSpec((NC, MM_BLK, D), lambda i: (0, i, 0)),
            pl.BlockSpec((MM_BLK, 1), lambda i: (i, 0)),
            pl.BlockSpec((D,), lambda i: (0,)),
        ],
        out_specs=pl.BlockSpec((MM_BLK, D), lambda i: (i, 0)),
        out_shape=jax.ShapeDtypeStruct((N_NODES, D), jnp.float32),
    )(parts, dis, b)


@functools.partial(
    pl.kernel,
    out_type=[
        jax.ShapeDtypeStruct((N_NODES,), jnp.float32),
        jax.ShapeDtypeStruct((N_NODES,), jnp.float32),
    ],
    mesh=_MESH,
    compiler_params=_SC_PARAMS,
    scratch_types=[
        pltpu.VMEM((BLK, CH), jnp.int32),
        pltpu.VMEM((BLK, CH), jnp.float32),
        pltpu.VMEM((N_NODES,), jnp.float32),
        pltpu.VMEM_SHARED((N_NODES,), jnp.float32),
    ],
)
def _sc_deg(col_hbm, ew_hbm, deg0_hbm, deg1_hbm, colb, ewb, zbuf, deg):
    cid = lax.axis_index("c")
    sid = lax.axis_index("s")

    zv = jnp.zeros((L,), jnp.float32)

    def _zb(i, _):
        zbuf[pl.ds(i * L, L)] = zv
        return 0
    lax.fori_loop(0, N_NODES // L, _zb, 0)

    @pl.when(sid == 0)
    def _():
        pltpu.sync_copy(zbuf, deg)

    plsc.subcore_barrier()

    def _dblk(t, _):
        dbase = cid * (NCHUNK // NC) + sid * CPW + t * BLK
        pltpu.sync_copy(col_hbm.at[pl.ds(dbase, BLK)], colb)
        pltpu.sync_copy(ew_hbm.at[pl.ds(dbase, BLK)], ewb)

        def _dadd(j, _):
            pltpu.sync_copy(ewb.at[j], deg.at[colb.at[j]], add=True)
            return 0
        lax.fori_loop(0, BLK, _dadd, 0)
        return 0
    lax.fori_loop(0, DNBLK, _dblk, 0)

    plsc.subcore_barrier()

    @pl.when(jnp.logical_and(cid == 0, sid == 0))
    def _():
        pltpu.sync_copy(deg, deg0_hbm)

    @pl.when(jnp.logical_and(cid == 1, sid == 0))
    def _():
        pltpu.sync_copy(deg, deg1_hbm)


@functools.partial(
    pl.kernel,
    out_type=jax.ShapeDtypeStruct((NC, N_NODES, D), jnp.float32),
    mesh=_MESH,
    compiler_params=_SC_PARAMS,
    scratch_types=[
        pltpu.VMEM((BLK, CH), jnp.int32),
        pltpu.VMEM((BLK, CH), jnp.int32),
        pltpu.VMEM((BLK, CH), jnp.float32),
        pltpu.VMEM((CH, D), jnp.float32),
        pltpu.VMEM((CH, D), jnp.float32),
        pltpu.VMEM_SHARED((N_NODES, D), jnp.float32),
        pltpu.SemaphoreType.DMA,
        pltpu.SemaphoreType.DMA,
    ],
)
def _sc_msg(g_hbm, row_hbm, col_hbm, ew_hbm, out_hbm,
            rowb, colb, ewb, msgA, msgB, acc, gsA, gsB):
    cid = lax.axis_index("c")
    sid = lax.axis_index("s")
    wid = cid * NS + sid

    zv = jnp.zeros((L,), jnp.float32)

    def _zmsg(i, _):
        for k in range(D // L):
            msgA[i, pl.ds(k * L, L)] = zv
        return 0
    lax.fori_loop(0, CH, _zmsg, 0)

    def _zacc(t, _):
        c = sid + t * NS
        @pl.when(c < NROWCH)
        def _():
            r = c * ROWB
            pltpu.sync_copy(msgA.at[pl.ds(0, ROWB)], acc.at[pl.ds(r, ROWB)])
        return 0
    lax.fori_loop(0, (NROWCH + NS - 1) // NS, _zacc, 0)

    plsc.subcore_barrier()

    def _scale(mref, j):
        jv = jnp.full((L,), j, jnp.int32)

        def _rloop(q, _):
            e0 = q * 4
            ns = []
            for u in range(4):
                ev = jnp.full((L,), e0 + u, jnp.int32)
                ns.append(plsc.load_gather(ewb, [jv, ev]))
            for k in range(D // L):
                sl = pl.ds(k * L, L)
                for u in range(4):
                    mref[e0 + u, sl] = mref[e0 + u, sl] * ns[u]
            return 0
        lax.fori_loop(0, CH // 4, _rloop, 0)

    def _mblk(t, _):
        wbase = wid * CPW + t * BLK
        pltpu.sync_copy(row_hbm.at[pl.ds(wbase, BLK)], rowb)
        pltpu.sync_copy(col_hbm.at[pl.ds(wbase, BLK)], colb)
        pltpu.sync_copy(ew_hbm.at[pl.ds(wbase, BLK)], ewb)

        pltpu.async_copy(g_hbm.at[rowb.at[0]], msgA, gsA)

        def _mpair(p, _):
            jA = 2 * p
            jB = 2 * p + 1

            pltpu.make_async_copy(g_hbm.at[rowb.at[jA]], msgA, gsA).wait()
            pltpu.async_copy(g_hbm.at[rowb.at[jB]], msgB, gsB)
            _scale(msgA, jA)
            pltpu.sync_copy(msgA, acc.at[colb.at[jA]], add=True)

            @pl.when(p < BLK // 2 - 1)
            def _():
                pltpu.async_copy(g_hbm.at[rowb.at[jA + 2]], msgA, gsA)

            pltpu.make_async_copy(g_hbm.at[rowb.at[jB]], msgB, gsB).wait()
            _scale(msgB, jB)
            pltpu.sync_copy(msgB, acc.at[colb.at[jB]], add=True)
            return 0
        lax.fori_loop(0, BLK // 2, _mpair, 0)
        return 0
    lax.fori_loop(0, MBLK, _mblk, 0)

    plsc.subcore_barrier()

    def _oloop(t, _):
        c = sid + t * NS
        @pl.when(c < NROWCH)
        def _():
            r = c * ROWB
            pltpu.sync_copy(acc.at[pl.ds(r, ROWB)], out_hbm.at[cid, pl.ds(r, ROWB), :])
        return 0
    lax.fori_loop(0, (NROWCH + NS - 1) // NS, _oloop, 0)


def kernel(x, edge_index, edge_weight, W, b):
    pad = E_PAD - N_EDGES
    zi = jnp.arange(pad, dtype=jnp.int32) % N_NODES
    row = jnp.concatenate([edge_index[0].astype(jnp.int32), zi]).reshape(NCHUNK, CH)
    col = jnp.concatenate([edge_index[1].astype(jnp.int32), zi]).reshape(NCHUNK, CH)
    ew = jnp.concatenate([edge_weight, jnp.zeros((pad,), jnp.float32)]).reshape(NCHUNK, CH)
    deg0, deg1 = _sc_deg(col, ew)
    g, dis = _g_and_dis(x, W, deg0.reshape(N_NODES, 1), deg1.reshape(N_NODES, 1))
    parts = _sc_msg(g, row, col, ew)
    return _final_add(parts, dis, b)


# speedup vs baseline: 1.1092x; 1.0030x over previous
import functools

import jax
import jax.numpy as jnp
from jax import lax
from jax.experimental import pallas as pl
from jax.experimental.pallas import tpu as pltpu
from jax.experimental.pallas import tpu_sc as plsc

N_NODES = 10000
N_EDGES = 320000
D = 128

NC = 2
NS = 16
L = 16
NW = NC * NS

CH = 128
NCHUNK = 2560
E_PAD = NCHUNK * CH
CPW = NCHUNK // NW
BLK = 16
MBLK = CPW // BLK
DNBLK = CPW // BLK

ROWB = 80
NROWCH = N_NODES // ROWB

MM_BLK = 400
MM_GRID = N_NODES // MM_BLK

_MESH = plsc.VectorSubcoreMesh(core_axis_name="c", subcore_axis_name="s")
_SC_PARAMS = pltpu.CompilerParams(needs_layout_passes=False)


def _mm_body(x_ref, w_ref, h_ref):
    h_ref[...] = jnp.dot(x_ref[...], w_ref[...],
                         preferred_element_type=jnp.float32)


def _matmul(x, W):
    return pl.pallas_call(
        _mm_body,
        grid=(MM_GRID,),
        in_specs=[
            pl.BlockSpec((MM_BLK, D), lambda i: (i, 0)),
            pl.BlockSpec((D, D), lambda i: (0, 0)),
        ],
        out_specs=pl.BlockSpec((MM_BLK, D), lambda i: (i, 0)),
        out_shape=jax.ShapeDtypeStruct((N_NODES, D), jnp.float32),
    )(x, W)


def _gk_body(h_ref, d0_ref, d1_ref, g_ref, dis_ref):
    d = d0_ref[:, 0] + d1_ref[:, 0]
    dis = jnp.where(d > 0.0, lax.rsqrt(jnp.where(d > 0.0, d, 1.0)), 0.0)
    dis_ref[...] = dis[:, None]
    g_ref[...] = h_ref[...] * dis[:, None]


def _g_and_dis(h, deg0, deg1):
    return pl.pallas_call(
        _gk_body,
        grid=(MM_GRID,),
        in_specs=[
            pl.BlockSpec((MM_BLK, D), lambda i: (i, 0)),
            pl.BlockSpec((MM_BLK, 1), lambda i: (i, 0)),
            pl.BlockSpec((MM_BLK, 1), lambda i: (i, 0)),
        ],
        out_specs=[
            pl.BlockSpec((MM_BLK, D), lambda i: (i, 0)),
            pl.BlockSpec((MM_BLK, 1), lambda i: (i, 0)),
        ],
        out_shape=[
            jax.ShapeDtypeStruct((N_NODES, D), jnp.float32),
            jax.ShapeDtypeStruct((N_NODES, 1), jnp.float32),
        ],
    )(h, deg0, deg1)


def _fin_body(p_ref, dis_ref, b_ref, o_ref):
    o_ref[...] = (p_ref[0] + p_ref[1]) * dis_ref[...] + b_ref[...]


def _final_add(parts, dis, b):
    return pl.pallas_call(
        _fin_body,
        grid=(MM_GRID,),
        in_specs=[
            pl.BlockSpec((NC, MM_BLK, D), lambda i: (0, i, 0)),
            pl.BlockSpec((MM_BLK, 1), lambda i: (i, 0)),
            pl.BlockSpec((D,), lambda i: (0,)),
        ],
        out_specs=pl.BlockSpec((MM_BLK, D), lambda i: (i, 0)),
        out_shape=jax.ShapeDtypeStruct((N_NODES, D), jnp.float32),
    )(parts, dis, b)


@functools.partial(
    pl.kernel,
    out_type=[
        jax.ShapeDtypeStruct((N_NODES,), jnp.float32),
        jax.ShapeDtypeStruct((N_NODES,), jnp.float32),
    ],
    mesh=_MESH,
    compiler_params=_SC_PARAMS,
    scratch_types=[
        pltpu.VMEM((BLK, CH), jnp.int32),
        pltpu.VMEM((BLK, CH), jnp.float32),
        pltpu.VMEM((N_NODES,), jnp.float32),
        pltpu.VMEM_SHARED((N_NODES,), jnp.float32),
    ],
)
def _sc_deg(col_hbm, ew_hbm, deg0_hbm, deg1_hbm, colb, ewb, zbuf, deg):
    cid = lax.axis_index("c")
    sid = lax.axis_index("s")

    zv = jnp.zeros((L,), jnp.float32)

    def _zb(i, _):
        zbuf[pl.ds(i * L, L)] = zv
        return 0
    lax.fori_loop(0, N_NODES // L, _zb, 0)

    @pl.when(sid == 0)
    def _():
        pltpu.sync_copy(zbuf, deg)

    plsc.subcore_barrier()

    def _dblk(t, _):
        dbase = cid * (NCHUNK // NC) + sid * CPW + t * BLK
        pltpu.sync_copy(col_hbm.at[pl.ds(dbase, BLK)], colb)
        pltpu.sync_copy(ew_hbm.at[pl.ds(dbase, BLK)], ewb)

        def _dadd(j, _):
            pltpu.sync_copy(ewb.at[j], deg.at[colb.at[j]], add=True)
            return 0
        lax.fori_loop(0, BLK, _dadd, 0)
        return 0
    lax.fori_loop(0, DNBLK, _dblk, 0)

    plsc.subcore_barrier()

    @pl.when(jnp.logical_and(cid == 0, sid == 0))
    def _():
        pltpu.sync_copy(deg, deg0_hbm)

    @pl.when(jnp.logical_and(cid == 1, sid == 0))
    def _():
        pltpu.sync_copy(deg, deg1_hbm)


@functools.partial(
    pl.kernel,
    out_type=jax.ShapeDtypeStruct((NC, N_NODES, D), jnp.float32),
    mesh=_MESH,
    compiler_params=_SC_PARAMS,
    scratch_types=[
        pltpu.VMEM((BLK, CH), jnp.int32),
        pltpu.VMEM((BLK, CH), jnp.int32),
        pltpu.VMEM((BLK, CH), jnp.float32),
        pltpu.VMEM((CH, D), jnp.float32),
        pltpu.VMEM((CH, D), jnp.float32),
        pltpu.VMEM_SHARED((N_NODES, D), jnp.float32),
        pltpu.SemaphoreType.DMA,
        pltpu.SemaphoreType.DMA,
    ],
)
def _sc_msg(g_hbm, row_hbm, col_hbm, ew_hbm, out_hbm,
            rowb, colb, ewb, msgA, msgB, acc, gsA, gsB):
    cid = lax.axis_index("c")
    sid = lax.axis_index("s")
    wid = cid * NS + sid

    zv = jnp.zeros((L,), jnp.float32)

    def _zmsg(i, _):
        for k in range(D // L):
            msgA[i, pl.ds(k * L, L)] = zv
        return 0
    lax.fori_loop(0, CH, _zmsg, 0)

    def _zacc(t, _):
        c = sid + t * NS
        @pl.when(c < NROWCH)
        def _():
            r = c * ROWB
            pltpu.sync_copy(msgA.at[pl.ds(0, ROWB)], acc.at[pl.ds(r, ROWB)])
        return 0
    lax.fori_loop(0, (NROWCH + NS - 1) // NS, _zacc, 0)

    plsc.subcore_barrier()

    def _scale(mref, j):
        jv = jnp.full((L,), j, jnp.int32)

        def _rloop(q, _):
            e0 = q * 4
            ns = []
            for u in range(4):
                ev = jnp.full((L,), e0 + u, jnp.int32)
                ns.append(plsc.load_gather(ewb, [jv, ev]))
            for k in range(D // L):
                sl = pl.ds(k * L, L)
                for u in range(4):
                    mref[e0 + u, sl] = mref[e0 + u, sl] * ns[u]
            return 0
        lax.fori_loop(0, CH // 4, _rloop, 0)

    def _mblk(t, _):
        wbase = wid * CPW + t * BLK
        pltpu.sync_copy(row_hbm.at[pl.ds(wbase, BLK)], rowb)
        pltpu.sync_copy(col_hbm.at[pl.ds(wbase, BLK)], colb)
        pltpu.sync_copy(ew_hbm.at[pl.ds(wbase, BLK)], ewb)

        pltpu.async_copy(g_hbm.at[rowb.at[0]], msgA, gsA)

        def _mpair(p, _):
            jA = 2 * p
            jB = 2 * p + 1

            pltpu.make_async_copy(g_hbm.at[rowb.at[jA]], msgA, gsA).wait()
            pltpu.async_copy(g_hbm.at[rowb.at[jB]], msgB, gsB)
            _scale(msgA, jA)
            pltpu.sync_copy(msgA, acc.at[colb.at[jA]], add=True)

            @pl.when(p < BLK // 2 - 1)
            def _():
                pltpu.async_copy(g_hbm.at[rowb.at[jA + 2]], msgA, gsA)

            pltpu.make_async_copy(g_hbm.at[rowb.at[jB]], msgB, gsB).wait()
            _scale(msgB, jB)
            pltpu.sync_copy(msgB, acc.at[colb.at[jB]], add=True)
            return 0
        lax.fori_loop(0, BLK // 2, _mpair, 0)
        return 0
    lax.fori_loop(0, MBLK, _mblk, 0)

    plsc.subcore_barrier()

    def _oloop(t, _):
        c = sid + t * NS
        @pl.when(c < NROWCH)
        def _():
            r = c * ROWB
            pltpu.sync_copy(acc.at[pl.ds(r, ROWB)], out_hbm.at[cid, pl.ds(r, ROWB), :])
        return 0
    lax.fori_loop(0, (NROWCH + NS - 1) // NS, _oloop, 0)


def kernel(x, edge_index, edge_weight, W, b):
    pad = E_PAD - N_EDGES
    zi = jnp.arange(pad, dtype=jnp.int32) % N_NODES
    row = jnp.concatenate([edge_index[0].astype(jnp.int32), zi]).reshape(NCHUNK, CH)
    col = jnp.concatenate([edge_index[1].astype(jnp.int32), zi]).reshape(NCHUNK, CH)
    ew = jnp.concatenate([edge_weight, jnp.zeros((pad,), jnp.float32)]).reshape(NCHUNK, CH)
    h = _matmul(x, W)
    deg0, deg1 = _sc_deg(col, ew)
    g, dis = _g_and_dis(h, deg0.reshape(N_NODES, 1), deg1.reshape(N_NODES, 1))
    parts = _sc_msg(g, row, col, ew)
    return _final_add(parts, dis, b)


# speedup vs baseline: 1.1634x; 1.0489x over previous
import functools

import jax
import jax.numpy as jnp
from jax import lax
from jax.experimental import pallas as pl
from jax.experimental.pallas import tpu as pltpu
from jax.experimental.pallas import tpu_sc as plsc

N_NODES = 10000
N_EDGES = 320000
D = 128

NC = 2
NS = 16
L = 16
NW = NC * NS

CH = 128
NCHUNK = 2560
E_PAD = NCHUNK * CH
CPW = NCHUNK // NW
BLK = 40
MBLK = CPW // BLK
DNBLK = CPW // BLK

ROWB = 80
NROWCH = N_NODES // ROWB

MM_BLK = 400
MM_GRID = N_NODES // MM_BLK

_MESH = plsc.VectorSubcoreMesh(core_axis_name="c", subcore_axis_name="s")
_SC_PARAMS = pltpu.CompilerParams(needs_layout_passes=False)


def _mm_body(x_ref, w_ref, h_ref):
    h_ref[...] = jnp.dot(x_ref[...], w_ref[...],
                         preferred_element_type=jnp.float32)


def _matmul(x, W):
    return pl.pallas_call(
        _mm_body,
        grid=(MM_GRID,),
        in_specs=[
            pl.BlockSpec((MM_BLK, D), lambda i: (i, 0)),
            pl.BlockSpec((D, D), lambda i: (0, 0)),
        ],
        out_specs=pl.BlockSpec((MM_BLK, D), lambda i: (i, 0)),
        out_shape=jax.ShapeDtypeStruct((N_NODES, D), jnp.float32),
    )(x, W)


def _gk_body(h_ref, d0_ref, d1_ref, g_ref, dis_ref):
    d = d0_ref[:, 0] + d1_ref[:, 0]
    dis = jnp.where(d > 0.0, lax.rsqrt(jnp.where(d > 0.0, d, 1.0)), 0.0)
    dis_ref[...] = dis[:, None]
    g_ref[...] = h_ref[...] * dis[:, None]


def _g_and_dis(h, deg0, deg1):
    return pl.pallas_call(
        _gk_body,
        grid=(MM_GRID,),
        in_specs=[
            pl.BlockSpec((MM_BLK, D), lambda i: (i, 0)),
            pl.BlockSpec((MM_BLK, 1), lambda i: (i, 0)),
            pl.BlockSpec((MM_BLK, 1), lambda i: (i, 0)),
        ],
        out_specs=[
            pl.BlockSpec((MM_BLK, D), lambda i: (i, 0)),
            pl.BlockSpec((MM_BLK, 1), lambda i: (i, 0)),
        ],
        out_shape=[
            jax.ShapeDtypeStruct((N_NODES, D), jnp.float32),
            jax.ShapeDtypeStruct((N_NODES, 1), jnp.float32),
        ],
    )(h, deg0, deg1)


def _fin_body(p_ref, dis_ref, b_ref, o_ref):
    o_ref[...] = (p_ref[0] + p_ref[1]) * dis_ref[...] + b_ref[...]


def _final_add(parts, dis, b):
    return pl.pallas_call(
        _fin_body,
        grid=(MM_GRID,),
        in_specs=[
            pl.BlockSpec((NC, MM_BLK, D), lambda i: (0, i, 0)),
            pl.BlockSpec((MM_BLK, 1), lambda i: (i, 0)),
            pl.BlockSpec((D,), lambda i: (0,)),
        ],
        out_specs=pl.BlockSpec((MM_BLK, D), lambda i: (i, 0)),
        out_shape=jax.ShapeDtypeStruct((N_NODES, D), jnp.float32),
    )(parts, dis, b)


@functools.partial(
    pl.kernel,
    out_type=[
        jax.ShapeDtypeStruct((N_NODES,), jnp.float32),
        jax.ShapeDtypeStruct((N_NODES,), jnp.float32),
    ],
    mesh=_MESH,
    compiler_params=_SC_PARAMS,
    scratch_types=[
        pltpu.VMEM((BLK, CH), jnp.int32),
        pltpu.VMEM((BLK, CH), jnp.float32),
        pltpu.VMEM((N_NODES,), jnp.float32),
        pltpu.VMEM_SHARED((N_NODES,), jnp.float32),
    ],
)
def _sc_deg(col_hbm, ew_hbm, deg0_hbm, deg1_hbm, colb, ewb, zbuf, deg):
    cid = lax.axis_index("c")
    sid = lax.axis_index("s")

    zv = jnp.zeros((L,), jnp.float32)

    def _zb(i, _):
        zbuf[pl.ds(i * L, L)] = zv
        return 0
    lax.fori_loop(0, N_NODES // L, _zb, 0)

    @pl.when(sid == 0)
    def _():
        pltpu.sync_copy(zbuf, deg)

    plsc.subcore_barrier()

    def _dblk(t, _):
        dbase = cid * (NCHUNK // NC) + sid * CPW + t * BLK
        pltpu.sync_copy(col_hbm.at[pl.ds(dbase, BLK)], colb)
        pltpu.sync_copy(ew_hbm.at[pl.ds(dbase, BLK)], ewb)

        def _dadd(j, _):
            pltpu.sync_copy(ewb.at[j], deg.at[colb.at[j]], add=True)
            return 0
        lax.fori_loop(0, BLK, _dadd, 0)
        return 0
    lax.fori_loop(0, DNBLK, _dblk, 0)

    plsc.subcore_barrier()

    @pl.when(jnp.logical_and(cid == 0, sid == 0))
    def _():
        pltpu.sync_copy(deg, deg0_hbm)

    @pl.when(jnp.logical_and(cid == 1, sid == 0))
    def _():
        pltpu.sync_copy(deg, deg1_hbm)


@functools.partial(
    pl.kernel,
    out_type=jax.ShapeDtypeStruct((NC, N_NODES, D), jnp.float32),
    mesh=_MESH,
    compiler_params=_SC_PARAMS,
    scratch_types=[
        pltpu.VMEM((BLK, CH), jnp.int32),
        pltpu.VMEM((BLK, CH), jnp.int32),
        pltpu.VMEM((BLK, CH), jnp.float32),
        pltpu.VMEM((CH, D), jnp.float32),
        pltpu.VMEM((CH, D), jnp.float32),
        pltpu.VMEM_SHARED((N_NODES, D), jnp.float32),
        pltpu.SemaphoreType.DMA,
        pltpu.SemaphoreType.DMA,
    ],
)
def _sc_msg(g_hbm, row_hbm, col_hbm, ew_hbm, out_hbm,
            rowb, colb, ewb, msgA, msgB, acc, gsA, gsB):
    cid = lax.axis_index("c")
    sid = lax.axis_index("s")
    wid = cid * NS + sid

    zv = jnp.zeros((L,), jnp.float32)

    def _zmsg(i, _):
        for k in range(D // L):
            msgA[i, pl.ds(k * L, L)] = zv
        return 0
    lax.fori_loop(0, CH, _zmsg, 0)

    def _zacc(t, _):
        c = sid + t * NS
        @pl.when(c < NROWCH)
        def _():
            r = c * ROWB
            pltpu.sync_copy(msgA.at[pl.ds(0, ROWB)], acc.at[pl.ds(r, ROWB)])
        return 0
    lax.fori_loop(0, (NROWCH + NS - 1) // NS, _zacc, 0)

    plsc.subcore_barrier()

    def _scale(mref, j):
        jv = jnp.full((L,), j, jnp.int32)

        def _rloop(q, _):
            e0 = q * 4
            ns = []
            for u in range(4):
                ev = jnp.full((L,), e0 + u, jnp.int32)
                ns.append(plsc.load_gather(ewb, [jv, ev]))
            for k in range(D // L):
                sl = pl.ds(k * L, L)
                for u in range(4):
                    mref[e0 + u, sl] = mref[e0 + u, sl] * ns[u]
            return 0
        lax.fori_loop(0, CH // 4, _rloop, 0)

    def _mblk(t, _):
        wbase = wid * CPW + t * BLK
        pltpu.sync_copy(row_hbm.at[pl.ds(wbase, BLK)], rowb)
        pltpu.sync_copy(col_hbm.at[pl.ds(wbase, BLK)], colb)
        pltpu.sync_copy(ew_hbm.at[pl.ds(wbase, BLK)], ewb)

        pltpu.async_copy(g_hbm.at[rowb.at[0]], msgA, gsA)

        def _mpair(p, _):
            jA = 2 * p
            jB = 2 * p + 1

            pltpu.make_async_copy(g_hbm.at[rowb.at[jA]], msgA, gsA).wait()
            pltpu.async_copy(g_hbm.at[rowb.at[jB]], msgB, gsB)
            _scale(msgA, jA)
            pltpu.sync_copy(msgA, acc.at[colb.at[jA]], add=True)

            @pl.when(p < BLK // 2 - 1)
            def _():
                pltpu.async_copy(g_hbm.at[rowb.at[jA + 2]], msgA, gsA)

            pltpu.make_async_copy(g_hbm.at[rowb.at[jB]], msgB, gsB).wait()
            _scale(msgB, jB)
            pltpu.sync_copy(msgB, acc.at[colb.at[jB]], add=True)
            return 0
        lax.fori_loop(0, BLK // 2, _mpair, 0)
        return 0
    lax.fori_loop(0, MBLK, _mblk, 0)

    plsc.subcore_barrier()

    def _oloop(t, _):
        c = sid + t * NS
        @pl.when(c < NROWCH)
        def _():
            r = c * ROWB
            pltpu.sync_copy(acc.at[pl.ds(r, ROWB)], out_hbm.at[cid, pl.ds(r, ROWB), :])
        return 0
    lax.fori_loop(0, (NROWCH + NS - 1) // NS, _oloop, 0)


def kernel(x, edge_index, edge_weight, W, b):
    pad = E_PAD - N_EDGES
    zi = jnp.arange(pad, dtype=jnp.int32) % N_NODES
    row = jnp.concatenate([edge_index[0].astype(jnp.int32), zi]).reshape(NCHUNK, CH)
    col = jnp.concatenate([edge_index[1].astype(jnp.int32), zi]).reshape(NCHUNK, CH)
    ew = jnp.concatenate([edge_weight, jnp.zeros((pad,), jnp.float32)]).reshape(NCHUNK, CH)
    h = _matmul(x, W)
    deg0, deg1 = _sc_deg(col, ew)
    g, dis = _g_and_dis(h, deg0.reshape(N_NODES, 1), deg1.reshape(N_NODES, 1))
    parts = _sc_msg(g, row, col, ew)
    return _final_add(parts, dis, b)
